# baseline (speedup 1.0000x reference)
import jax
import jax.numpy as jnp
from jax.experimental import pallas as pl


def _block_rows(n, target):
    best = None
    for b in range(8, min(n, target) + 1, 8):
        if n % b == 0:
            best = b
    return best if best is not None else n


def _proj_body(x_ref, wg_ref, bg_ref, w1_ref, b1_ref, g0_ref, h0_ref):
    x = x_ref[...]
    g0_ref[...] = jnp.dot(x, wg_ref[...], preferred_element_type=jnp.float32) + bg_ref[...]
    h0_ref[...] = jnp.dot(x, w1_ref[...], preferred_element_type=jnp.float32) + b1_ref[...]


def _hyper_body(hh_ref, h0_ref, w2_ref, b2_ref, he_ref, z_ref):
    acc = jnp.dot(hh_ref[...], h0_ref[...], preferred_element_type=jnp.float32)
    he = jnp.maximum(acc, 0.0)
    he_ref[...] = he
    z_ref[...] = jnp.dot(he, w2_ref[...], preferred_element_type=jnp.float32) + b2_ref[...]


def _main_body(gg_ref, hg_ref, x_ref, g0_ref, z_ref, wm_ref, bm_ref, fused_ref):
    d = x_ref.shape[1]
    gnn = jnp.maximum(
        jnp.dot(gg_ref[...], g0_ref[...], preferred_element_type=jnp.float32), 0.0)
    hgn = jnp.maximum(
        jnp.dot(hg_ref[...], z_ref[...], preferred_element_type=jnp.float32), 0.0)
    fused = jnp.dot(x_ref[...], wm_ref[0:d, :], preferred_element_type=jnp.float32)
    fused += jnp.dot(gnn, wm_ref[d:2 * d, :], preferred_element_type=jnp.float32)
    fused += jnp.dot(hgn, wm_ref[2 * d:3 * d, :], preferred_element_type=jnp.float32)
    fused_ref[...] = fused + bm_ref[...]


def kernel(x, norm_GG, norm_HH, norm_HG, Wg, bg, W1, b1, W2, b2, Wm, bm):
    n, d = x.shape
    h = norm_HH.shape[0]
    f32 = jnp.float32
    bg2 = bg.reshape(1, d)
    b12 = b1.reshape(1, d)
    b22 = b2.reshape(1, d)
    bm2 = bm.reshape(1, d)

    bm_n = _block_rows(n, 400)
    bm_h = _block_rows(h, 200)
    bm_main = _block_rows(n, 200)

    g0, h0 = pl.pallas_call(
        _proj_body,
        grid=(n // bm_n,),
        in_specs=[
            pl.BlockSpec((bm_n, d), lambda i: (i, 0)),
            pl.BlockSpec((d, d), lambda i: (0, 0)),
            pl.BlockSpec((1, d), lambda i: (0, 0)),
            pl.BlockSpec((d, d), lambda i: (0, 0)),
            pl.BlockSpec((1, d), lambda i: (0, 0)),
        ],
        out_specs=[
            pl.BlockSpec((bm_n, d), lambda i: (i, 0)),
            pl.BlockSpec((bm_n, d), lambda i: (i, 0)),
        ],
        out_shape=[
            jax.ShapeDtypeStruct((n, d), f32),
            jax.ShapeDtypeStruct((n, d), f32),
        ],
    )(x, Wg, bg2, W1, b12)

    hyper_emb, z = pl.pallas_call(
        _hyper_body,
        grid=(h // bm_h,),
        in_specs=[
            pl.BlockSpec((bm_h, n), lambda i: (i, 0)),
            pl.BlockSpec((n, d), lambda i: (0, 0)),
            pl.BlockSpec((d, d), lambda i: (0, 0)),
            pl.BlockSpec((1, d), lambda i: (0, 0)),
        ],
        out_specs=[
            pl.BlockSpec((bm_h, d), lambda i: (i, 0)),
            pl.BlockSpec((bm_h, d), lambda i: (i, 0)),
        ],
        out_shape=[
            jax.ShapeDtypeStruct((h, d), f32),
            jax.ShapeDtypeStruct((h, d), f32),
        ],
    )(norm_HH, h0, W2, b22)

    fused_emb = pl.pallas_call(
        _main_body,
        grid=(n // bm_main,),
        in_specs=[
            pl.BlockSpec((bm_main, n), lambda i: (i, 0)),
            pl.BlockSpec((bm_main, h), lambda i: (i, 0)),
            pl.BlockSpec((bm_main, d), lambda i: (i, 0)),
            pl.BlockSpec((n, d), lambda i: (0, 0)),
            pl.BlockSpec((h, d), lambda i: (0, 0)),
            pl.BlockSpec((3 * d, d), lambda i: (0, 0)),
            pl.BlockSpec((1, d), lambda i: (0, 0)),
        ],
        out_specs=pl.BlockSpec((bm_main, d), lambda i: (i, 0)),
        out_shape=jax.ShapeDtypeStruct((n, d), f32),
    )(norm_GG, norm_HG, x, g0, z, Wm, bm2)

    return (fused_emb, hyper_emb)
